# baseline (speedup 1.0000x reference)
import functools

import jax
import jax.numpy as jnp
from jax import lax
from jax.experimental import pallas as pl
from jax.experimental.pallas import tpu as pltpu
from jax.experimental.pallas import tpu_sc as plsc

N = 10000
E = 320000
D = 128
DOUT = 64

NC = 2
NS = 16
NW = NC * NS
K = 125
NCH = (E // NW) // K
EPT_DEG = E // NS
NROW = N // 16

_MESH = plsc.VectorSubcoreMesh(core_axis_name="c", subcore_axis_name="s")


def _zero16():
    return jnp.zeros((16,), jnp.float32)


def _prep_body(col_h, ew_h, row_h, dix_h, w_h, dis_h, inv_h,
               colv, ewv, degf, buf625, invb, wbuf, dixv, deg_sh):
    c = lax.axis_index("c")
    s = lax.axis_index("s")
    wid = s * NC + c

    def zloop(i, _):
        degf[pl.ds(i * 16, 16)] = _zero16()
        buf625[i] = _zero16()
        return 0
    lax.fori_loop(0, NROW, zloop, 0)

    @pl.when(s == 0)
    def _():
        pltpu.sync_copy(buf625, deg_sh)

    pltpu.sync_copy(dix_h, dixv)

    base_e = s * EPT_DEG
    pltpu.sync_copy(col_h.at[pl.ds(base_e, EPT_DEG)], colv)
    pltpu.sync_copy(ew_h.at[pl.ds(base_e, EPT_DEG)], ewv)

    plsc.subcore_barrier()

    def dloop(i, _):
        off = i * 16
        cvec = colv[pl.ds(off, 16)]
        evec = ewv[pl.ds(off, 16)]
        plsc.addupdate_scatter(degf, [cvec], evec)
        return 0
    lax.fori_loop(0, EPT_DEG // 16, dloop, 0)

    def cpl(i, _):
        buf625[i] = degf[pl.ds(i * 16, 16)]
        return 0
    lax.fori_loop(0, NROW, cpl, 0)

    def closs(j, _):
        pltpu.sync_copy(buf625.at[pl.ds(j * K, K)],
                        deg_sh.at[dixv.at[j]], add=True)
        return 0
    lax.fori_loop(0, NROW // K, closs, 0)

    plsc.subcore_barrier()

    pltpu.sync_copy(deg_sh, buf625)

    def nloop(i, _):
        d = buf625[i] + 1.0
        iv = plsc.bitcast(d, jnp.int32)
        iv = jnp.int32(0x5F3759DF) - lax.shift_right_arithmetic(iv, 1)
        y = plsc.bitcast(iv, jnp.float32)
        y = y * (1.5 - 0.5 * d * y * y)
        y = y * (1.5 - 0.5 * d * y * y)
        y = y * (1.5 - 0.5 * d * y * y)
        buf625[i] = y
        degf[pl.ds(i * 16, 16)] = y
        return 0
    lax.fori_loop(0, NROW, nloop, 0)

    @pl.when(wid == 0)
    def _():
        def iloop(i, _):
            y = buf625[i]
            invb[i] = y * y
            return 0
        lax.fori_loop(0, NROW, iloop, 0)
        pltpu.sync_copy(buf625, dis_h)
        pltpu.sync_copy(invb, inv_h)

    base_w = wid * (E // NW)
    pltpu.sync_copy(row_h.at[pl.ds(base_w, E // NW)], colv.at[pl.ds(0, E // NW)])
    pltpu.sync_copy(ew_h.at[pl.ds(base_w, E // NW)], ewv.at[pl.ds(0, E // NW)])

    def wloop(i, _):
        off = i * 16
        rvec = colv[pl.ds(off, 16)]
        dvec = plsc.load_gather(degf, [rvec])
        wbuf[i] = dvec * ewv[pl.ds(off, 16)]
        return 0
    lax.fori_loop(0, (E // NW) // 16, wloop, 0)

    pltpu.sync_copy(wbuf, w_h.at[wid])


_prep = functools.partial(
    pl.kernel,
    out_type=[
        jax.ShapeDtypeStruct((NW, NROW, 16), jnp.float32),
        jax.ShapeDtypeStruct((NROW, 16), jnp.float32),
        jax.ShapeDtypeStruct((NROW, 16), jnp.float32),
    ],
    mesh=_MESH,
    compiler_params=pltpu.CompilerParams(needs_layout_passes=False, use_tc_tiling_on_sc=False),
    scratch_types=[
        pltpu.VMEM((EPT_DEG,), jnp.int32),
        pltpu.VMEM((EPT_DEG,), jnp.float32),
        pltpu.VMEM((N,), jnp.float32),
        pltpu.VMEM((NROW, 16), jnp.float32),
        pltpu.VMEM((NROW, 16), jnp.float32),
        pltpu.VMEM((NROW, 16), jnp.float32),
        pltpu.VMEM((NROW // K, K), jnp.int32),
        pltpu.MemorySpace.VMEM_SHARED((NROW, 16), jnp.float32),
    ],
)(_prep_body)


def _agg_body(h_h, row_h, col_h, w_h, acc_h,
              r0, c0, w0, r1, c1, w1, r2, c2, w2,
              rows0, rows1, rows2,
              sem0, sem1, sem2, ssem0, ssem1, ssem2, acc_sh):
    c = lax.axis_index("c")
    s = lax.axis_index("s")
    wid = s * NC + c

    def zloop(i, _):
        for t in range(8):
            rows0[i, pl.ds(t * 16, 16)] = _zero16()
        return 0
    lax.fori_loop(0, K, zloop, 0)

    base = s * (N // NS)

    def zs(j, _):
        pltpu.sync_copy(rows0, acc_sh.at[pl.ds(base + j * K, K)])
        return 0
    lax.fori_loop(0, (N // NS) // K, zs, 0)

    plsc.subcore_barrier()

    def _scale(rows, wb):
        def edge(i, _):
            for u in range(5):
                k = i * 5 + u
                kv = jnp.zeros((16,), jnp.int32) + k
                sv = plsc.load_gather(wb, [kv])
                for t in range(8):
                    sl = pl.ds(t * 16, 16)
                    rows[k, sl] = rows[k, sl] * sv
            return 0
        lax.fori_loop(0, K // 5, edge, 0)

    def _wait(rows, sem):
        pltpu.make_async_copy(h_h.at[pl.ds(0, K)], rows, sem).wait()

    bufs = ((r0, c0, w0, rows0, sem0, ssem0),
            (r1, c1, w1, rows1, sem1, ssem1),
            (r2, c2, w2, rows2, sem2, ssem2))

    def _proc(rb, cb, wb, rows, gsem, ssm):
        _wait(rows, gsem)
        _scale(rows, wb)
        pltpu.async_copy(rows, acc_sh.at[cb], ssm, add=True)

    def _refill(j2, rb, cb, wb, rows, gsem, ssm, drain):
        if drain:
            _wait(rows, ssm)
        pltpu.sync_copy(row_h.at[wid, j2], rb)
        pltpu.sync_copy(col_h.at[wid, j2], cb)
        pltpu.sync_copy(w_h.at[wid, j2], wb)
        pltpu.async_copy(h_h.at[rb], rows, gsem)

    pltpu.sync_copy(row_h.at[wid, 0], r0)
    pltpu.sync_copy(col_h.at[wid, 0], c0)
    pltpu.sync_copy(w_h.at[wid, 0], w0)
    pltpu.async_copy(h_h.at[r0], rows0, sem0)
    pltpu.sync_copy(row_h.at[wid, 1], r1)
    pltpu.sync_copy(col_h.at[wid, 1], c1)
    pltpu.sync_copy(w_h.at[wid, 1], w1)
    pltpu.async_copy(h_h.at[r1], rows1, sem1)
    _proc(*bufs[0])
    _refill(2, *bufs[2], drain=False)

    def body(i, _):
        for u in range(3):
            j = 3 * i + 1 + u
            b = (1 + u) % 3
            _proc(*bufs[b])
            rb = (b + 2) % 3

            @pl.when(j < NCH - 2)
            def _():
                _refill(j + 2, *bufs[rb], drain=True)
        return 0
    lax.fori_loop(0, (NCH - 2) // 3, body, 0)

    _proc(*bufs[(NCH - 1) % 3])
    _wait(rows0, ssem0)
    _wait(rows1, ssem1)
    _wait(rows2, ssem2)

    plsc.subcore_barrier()

    pltpu.sync_copy(acc_sh.at[pl.ds(base, N // NS)],
                    acc_h.at[c, pl.ds(base, N // NS)])


_agg = functools.partial(
    pl.kernel,
    out_type=jax.ShapeDtypeStruct((NC, N, D), jnp.float32),
    mesh=_MESH,
    compiler_params=pltpu.CompilerParams(needs_layout_passes=False, use_tc_tiling_on_sc=False),
    scratch_types=[
        pltpu.VMEM((K,), jnp.int32),
        pltpu.VMEM((K,), jnp.int32),
        pltpu.VMEM((K,), jnp.float32),
        pltpu.VMEM((K,), jnp.int32),
        pltpu.VMEM((K,), jnp.int32),
        pltpu.VMEM((K,), jnp.float32),
        pltpu.VMEM((K,), jnp.int32),
        pltpu.VMEM((K,), jnp.int32),
        pltpu.VMEM((K,), jnp.float32),
        pltpu.VMEM((K, D), jnp.float32),
        pltpu.VMEM((K, D), jnp.float32),
        pltpu.VMEM((K, D), jnp.float32),
        pltpu.SemaphoreType.DMA,
        pltpu.SemaphoreType.DMA,
        pltpu.SemaphoreType.DMA,
        pltpu.SemaphoreType.DMA,
        pltpu.SemaphoreType.DMA,
        pltpu.SemaphoreType.DMA,
        pltpu.MemorySpace.VMEM_SHARED((N, D), jnp.float32),
    ],
)(_agg_body)


_RB = 1000


def _mm_body(x_ref, w_ref, o_ref):
    o_ref[...] = jnp.dot(x_ref[...], w_ref[...],
                         preferred_element_type=jnp.float32)


def _mm(x, W):
    return pl.pallas_call(
        _mm_body,
        grid=(N // _RB,),
        in_specs=[
            pl.BlockSpec((_RB, D), lambda i: (i, 0)),
            pl.BlockSpec((D, D), lambda i: (0, 0)),
        ],
        out_specs=pl.BlockSpec((_RB, D), lambda i: (i, 0)),
        out_shape=jax.ShapeDtypeStruct((N, D), jnp.float32),
    )(x, W)


def _emm_body(acc_ref, h_ref, dis_ref, inv_ref, b_ref, w_ref, o_ref):
    xn = jax.nn.relu(dis_ref[...] * (acc_ref[0] + acc_ref[1])
                     + inv_ref[...] * h_ref[...] + b_ref[...])
    o_ref[...] = jnp.dot(xn, w_ref[...], preferred_element_type=jnp.float32)


def _emm(acc, h, dis2, inv2, b, W):
    return pl.pallas_call(
        _emm_body,
        grid=(N // _RB,),
        in_specs=[
            pl.BlockSpec((NC, _RB, D), lambda i: (0, i, 0)),
            pl.BlockSpec((_RB, D), lambda i: (i, 0)),
            pl.BlockSpec((_RB, 1), lambda i: (i, 0)),
            pl.BlockSpec((_RB, 1), lambda i: (i, 0)),
            pl.BlockSpec((1, D), lambda i: (0, 0)),
            pl.BlockSpec((D, D), lambda i: (0, 0)),
        ],
        out_specs=pl.BlockSpec((_RB, D), lambda i: (i, 0)),
        out_shape=jax.ShapeDtypeStruct((N, D), jnp.float32),
    )(acc, h, dis2, inv2, b, W)


def _final_body(acc_ref, h_ref, dis_ref, inv_ref, b_ref, w_ref, bf_ref, o_ref):
    xn = jax.nn.relu(dis_ref[...] * (acc_ref[0] + acc_ref[1])
                     + inv_ref[...] * h_ref[...] + b_ref[...])
    logits = jnp.dot(xn, w_ref[...], preferred_element_type=jnp.float32)
    logits = logits + bf_ref[...]
    m = jnp.max(logits, axis=1, keepdims=True)
    sh = logits - m
    lse = jnp.log(jnp.sum(jnp.exp(sh), axis=1, keepdims=True))
    o_ref[...] = sh - lse


def _final(acc, h, dis2, inv2, b, Wfc, bfc):
    return pl.pallas_call(
        _final_body,
        grid=(N // _RB,),
        in_specs=[
            pl.BlockSpec((NC, _RB, D), lambda i: (0, i, 0)),
            pl.BlockSpec((_RB, D), lambda i: (i, 0)),
            pl.BlockSpec((_RB, 1), lambda i: (i, 0)),
            pl.BlockSpec((_RB, 1), lambda i: (i, 0)),
            pl.BlockSpec((1, D), lambda i: (0, 0)),
            pl.BlockSpec((D, DOUT), lambda i: (0, 0)),
            pl.BlockSpec((1, DOUT), lambda i: (0, 0)),
        ],
        out_specs=pl.BlockSpec((_RB, DOUT), lambda i: (i, 0)),
        out_shape=jax.ShapeDtypeStruct((N, DOUT), jnp.float32),
    )(acc, h, dis2, inv2, b, Wfc, bfc)


def kernel(x, edge_index, edge_attr, W1, b1, W2, b2, Wfc, bfc):
    row = edge_index[0].astype(jnp.int32)
    col = edge_index[1].astype(jnp.int32)
    ew = edge_attr.astype(jnp.float32)
    dix = jnp.arange(NROW, dtype=jnp.int32).reshape(NROW // K, K)

    h1 = _mm(x, W1)
    w3, dis625, inv625 = _prep(col, ew, row, dix)
    dis2 = dis625.reshape(N, 1)
    inv2 = inv625.reshape(N, 1)
    row2 = row.reshape(NW, NCH, K)
    col2 = col.reshape(NW, NCH, K)
    wch = w3.reshape(NW, NCH, K)

    acc1 = _agg(h1, row2, col2, wch)
    h2 = _emm(acc1, h1, dis2, inv2, b1.reshape(1, D), W2)
    acc2 = _agg(h2, row2, col2, wch)
    return _final(acc2, h2, dis2, inv2, b2.reshape(1, D), Wfc,
                  bfc.reshape(1, DOUT))


# speedup vs baseline: 1.2441x; 1.2441x over previous
import functools

import jax
import jax.numpy as jnp
from jax import lax
from jax.experimental import pallas as pl
from jax.experimental.pallas import tpu as pltpu
from jax.experimental.pallas import tpu_sc as plsc

N = 10000
E = 320000
D = 128
DOUT = 64

NC = 2
NS = 16
NW = NC * NS
K = 125
NCH = (E // NW) // K
EPT_DEG = E // NS
NROW = N // 16

_MESH = plsc.VectorSubcoreMesh(core_axis_name="c", subcore_axis_name="s")


def _zero16():
    return jnp.zeros((16,), jnp.float32)


def _prep_body(col_h, ew_h, row_h, dix_h, w_h, dis_h, inv_h,
               colv, ewv, degf, buf625, invb, wbuf, dixv, deg_sh):
    c = lax.axis_index("c")
    s = lax.axis_index("s")
    wid = s * NC + c

    def zloop(i, _):
        degf[pl.ds(i * 16, 16)] = _zero16()
        buf625[i] = _zero16()
        return 0
    lax.fori_loop(0, NROW, zloop, 0)

    @pl.when(s == 0)
    def _():
        pltpu.sync_copy(buf625, deg_sh)

    pltpu.sync_copy(dix_h, dixv)

    base_e = s * EPT_DEG
    pltpu.sync_copy(col_h.at[pl.ds(base_e, EPT_DEG)], colv)
    pltpu.sync_copy(ew_h.at[pl.ds(base_e, EPT_DEG)], ewv)

    plsc.subcore_barrier()

    def dloop(i, _):
        off = i * 16
        cvec = colv[pl.ds(off, 16)]
        evec = ewv[pl.ds(off, 16)]
        plsc.addupdate_scatter(degf, [cvec], evec)
        return 0
    lax.fori_loop(0, EPT_DEG // 16, dloop, 0)

    def cpl(i, _):
        buf625[i] = degf[pl.ds(i * 16, 16)]
        return 0
    lax.fori_loop(0, NROW, cpl, 0)

    def closs(j, _):
        pltpu.sync_copy(buf625.at[pl.ds(j * K, K)],
                        deg_sh.at[dixv.at[j]], add=True)
        return 0
    lax.fori_loop(0, NROW // K, closs, 0)

    plsc.subcore_barrier()

    pltpu.sync_copy(deg_sh, buf625)

    def nloop(i, _):
        d = buf625[i] + 1.0
        iv = plsc.bitcast(d, jnp.int32)
        iv = jnp.int32(0x5F3759DF) - lax.shift_right_arithmetic(iv, 1)
        y = plsc.bitcast(iv, jnp.float32)
        y = y * (1.5 - 0.5 * d * y * y)
        y = y * (1.5 - 0.5 * d * y * y)
        y = y * (1.5 - 0.5 * d * y * y)
        buf625[i] = y
        degf[pl.ds(i * 16, 16)] = y
        return 0
    lax.fori_loop(0, NROW, nloop, 0)

    @pl.when(wid == 0)
    def _():
        def iloop(i, _):
            y = buf625[i]
            invb[i] = y * y
            return 0
        lax.fori_loop(0, NROW, iloop, 0)
        pltpu.sync_copy(buf625, dis_h)
        pltpu.sync_copy(invb, inv_h)

    base_w = wid * (E // NW)
    pltpu.sync_copy(row_h.at[pl.ds(base_w, E // NW)], colv.at[pl.ds(0, E // NW)])
    pltpu.sync_copy(ew_h.at[pl.ds(base_w, E // NW)], ewv.at[pl.ds(0, E // NW)])

    def wloop(i, _):
        off = i * 16
        rvec = colv[pl.ds(off, 16)]
        dvec = plsc.load_gather(degf, [rvec])
        wbuf[i] = dvec * ewv[pl.ds(off, 16)]
        return 0
    lax.fori_loop(0, (E // NW) // 16, wloop, 0)

    pltpu.sync_copy(wbuf, w_h.at[wid])


_prep = functools.partial(
    pl.kernel,
    out_type=[
        jax.ShapeDtypeStruct((NW, NROW, 16), jnp.float32),
        jax.ShapeDtypeStruct((NROW, 16), jnp.float32),
        jax.ShapeDtypeStruct((NROW, 16), jnp.float32),
    ],
    mesh=_MESH,
    compiler_params=pltpu.CompilerParams(needs_layout_passes=False, use_tc_tiling_on_sc=False),
    scratch_types=[
        pltpu.VMEM((EPT_DEG,), jnp.int32),
        pltpu.VMEM((EPT_DEG,), jnp.float32),
        pltpu.VMEM((N,), jnp.float32),
        pltpu.VMEM((NROW, 16), jnp.float32),
        pltpu.VMEM((NROW, 16), jnp.float32),
        pltpu.VMEM((NROW, 16), jnp.float32),
        pltpu.VMEM((NROW // K, K), jnp.int32),
        pltpu.MemorySpace.VMEM_SHARED((NROW, 16), jnp.float32),
    ],
)(_prep_body)


def _agg_body(h_h, rcw_h, acc_h, rcw0, rcw1, rcw2, rows0, rows1, rows2,
              sem0, sem1, sem2, ssem0, ssem1, ssem2, acc_sh):
    c = lax.axis_index("c")
    s = lax.axis_index("s")
    wid = s * NC + c

    def zloop(i, _):
        for t in range(8):
            rows0[i, pl.ds(t * 16, 16)] = _zero16()
        return 0
    lax.fori_loop(0, K, zloop, 0)

    base = s * (N // NS)

    def zs(j, _):
        pltpu.sync_copy(rows0, acc_sh.at[pl.ds(base + j * K, K)])
        return 0
    lax.fori_loop(0, (N // NS) // K, zs, 0)

    plsc.subcore_barrier()

    two = jnp.zeros((16,), jnp.int32) + 2

    def _scale(rows, rcw):
        def edge(i, _):
            for u in range(5):
                k = i * 5 + u
                kv = jnp.zeros((16,), jnp.int32) + k
                sv = plsc.bitcast(plsc.load_gather(rcw, [two, kv]), jnp.float32)
                for t in range(8):
                    sl = pl.ds(t * 16, 16)
                    rows[k, sl] = rows[k, sl] * sv
            return 0
        lax.fori_loop(0, K // 5, edge, 0)

    def _wait(rows, sem):
        pltpu.make_async_copy(h_h.at[pl.ds(0, K)], rows, sem).wait()

    bufs = ((rcw0, rows0, sem0, ssem0),
            (rcw1, rows1, sem1, ssem1),
            (rcw2, rows2, sem2, ssem2))

    def _proc(rcw, rows, gsem, ssm):
        _wait(rows, gsem)
        _scale(rows, rcw)
        pltpu.async_copy(rows, acc_sh.at[rcw.at[1]], ssm, add=True)

    def _refill(j2, rcw, rows, gsem, ssm, drain):
        if drain:
            _wait(rows, ssm)
        pltpu.sync_copy(rcw_h.at[wid, j2], rcw)
        pltpu.async_copy(h_h.at[rcw.at[0]], rows, gsem)

    pltpu.sync_copy(rcw_h.at[wid, 0], rcw0)
    pltpu.async_copy(h_h.at[rcw0.at[0]], rows0, sem0)
    pltpu.sync_copy(rcw_h.at[wid, 1], rcw1)
    pltpu.async_copy(h_h.at[rcw1.at[0]], rows1, sem1)
    _proc(*bufs[0])
    _refill(2, *bufs[2], drain=False)

    def body(i, _):
        for u in range(3):
            j = 3 * i + 1 + u
            b = (1 + u) % 3
            _proc(*bufs[b])
            rb = (b + 2) % 3

            @pl.when(j < NCH - 2)
            def _():
                _refill(j + 2, *bufs[rb], drain=True)
        return 0
    lax.fori_loop(0, (NCH - 2) // 3, body, 0)

    _proc(*bufs[(NCH - 1) % 3])
    _wait(rows0, ssem0)
    _wait(rows1, ssem1)
    _wait(rows2, ssem2)

    plsc.subcore_barrier()

    pltpu.sync_copy(acc_sh.at[pl.ds(base, N // NS)],
                    acc_h.at[c, pl.ds(base, N // NS)])


_agg = functools.partial(
    pl.kernel,
    out_type=jax.ShapeDtypeStruct((NC, N, D), jnp.float32),
    mesh=_MESH,
    compiler_params=pltpu.CompilerParams(needs_layout_passes=False, use_tc_tiling_on_sc=False),
    scratch_types=[
        pltpu.VMEM((3, K), jnp.int32),
        pltpu.VMEM((3, K), jnp.int32),
        pltpu.VMEM((3, K), jnp.int32),
        pltpu.VMEM((K, D), jnp.float32),
        pltpu.VMEM((K, D), jnp.float32),
        pltpu.VMEM((K, D), jnp.float32),
        pltpu.SemaphoreType.DMA,
        pltpu.SemaphoreType.DMA,
        pltpu.SemaphoreType.DMA,
        pltpu.SemaphoreType.DMA,
        pltpu.SemaphoreType.DMA,
        pltpu.SemaphoreType.DMA,
        pltpu.MemorySpace.VMEM_SHARED((N, D), jnp.float32),
    ],
)(_agg_body)


_RB = 1000


def _mm_body(x_ref, w_ref, o_ref):
    o_ref[...] = jnp.dot(x_ref[...], w_ref[...],
                         preferred_element_type=jnp.float32)


def _mm(x, W):
    return pl.pallas_call(
        _mm_body,
        grid=(N // _RB,),
        in_specs=[
            pl.BlockSpec((_RB, D), lambda i: (i, 0)),
            pl.BlockSpec((D, D), lambda i: (0, 0)),
        ],
        out_specs=pl.BlockSpec((_RB, D), lambda i: (i, 0)),
        out_shape=jax.ShapeDtypeStruct((N, D), jnp.float32),
    )(x, W)


def _emm_body(acc_ref, h_ref, dis_ref, inv_ref, b_ref, w_ref, o_ref):
    xn = jax.nn.relu(dis_ref[...] * (acc_ref[0] + acc_ref[1])
                     + inv_ref[...] * h_ref[...] + b_ref[...])
    o_ref[...] = jnp.dot(xn, w_ref[...], preferred_element_type=jnp.float32)


def _emm(acc, h, dis2, inv2, b, W):
    return pl.pallas_call(
        _emm_body,
        grid=(N // _RB,),
        in_specs=[
            pl.BlockSpec((NC, _RB, D), lambda i: (0, i, 0)),
            pl.BlockSpec((_RB, D), lambda i: (i, 0)),
            pl.BlockSpec((_RB, 1), lambda i: (i, 0)),
            pl.BlockSpec((_RB, 1), lambda i: (i, 0)),
            pl.BlockSpec((1, D), lambda i: (0, 0)),
            pl.BlockSpec((D, D), lambda i: (0, 0)),
        ],
        out_specs=pl.BlockSpec((_RB, D), lambda i: (i, 0)),
        out_shape=jax.ShapeDtypeStruct((N, D), jnp.float32),
    )(acc, h, dis2, inv2, b, W)


def _final_body(acc_ref, h_ref, dis_ref, inv_ref, b_ref, w_ref, bf_ref, o_ref):
    xn = jax.nn.relu(dis_ref[...] * (acc_ref[0] + acc_ref[1])
                     + inv_ref[...] * h_ref[...] + b_ref[...])
    logits = jnp.dot(xn, w_ref[...], preferred_element_type=jnp.float32)
    logits = logits + bf_ref[...]
    m = jnp.max(logits, axis=1, keepdims=True)
    sh = logits - m
    lse = jnp.log(jnp.sum(jnp.exp(sh), axis=1, keepdims=True))
    o_ref[...] = sh - lse


def _final(acc, h, dis2, inv2, b, Wfc, bfc):
    return pl.pallas_call(
        _final_body,
        grid=(N // _RB,),
        in_specs=[
            pl.BlockSpec((NC, _RB, D), lambda i: (0, i, 0)),
            pl.BlockSpec((_RB, D), lambda i: (i, 0)),
            pl.BlockSpec((_RB, 1), lambda i: (i, 0)),
            pl.BlockSpec((_RB, 1), lambda i: (i, 0)),
            pl.BlockSpec((1, D), lambda i: (0, 0)),
            pl.BlockSpec((D, DOUT), lambda i: (0, 0)),
            pl.BlockSpec((1, DOUT), lambda i: (0, 0)),
        ],
        out_specs=pl.BlockSpec((_RB, DOUT), lambda i: (i, 0)),
        out_shape=jax.ShapeDtypeStruct((N, DOUT), jnp.float32),
    )(acc, h, dis2, inv2, b, Wfc, bfc)


def kernel(x, edge_index, edge_attr, W1, b1, W2, b2, Wfc, bfc):
    row = edge_index[0].astype(jnp.int32)
    col = edge_index[1].astype(jnp.int32)
    ew = edge_attr.astype(jnp.float32)
    dix = jnp.arange(NROW, dtype=jnp.int32).reshape(NROW // K, K)

    h1 = _mm(x, W1)
    w3, dis625, inv625 = _prep(col, ew, row, dix)
    dis2 = dis625.reshape(N, 1)
    inv2 = inv625.reshape(N, 1)
    rcw = jnp.stack([row.reshape(NW, NCH, K),
                     col.reshape(NW, NCH, K),
                     lax.bitcast_convert_type(w3, jnp.int32).reshape(NW, NCH, K)],
                    axis=2)

    acc1 = _agg(h1, rcw)
    h2 = _emm(acc1, h1, dis2, inv2, b1.reshape(1, D), W2)
    acc2 = _agg(h2, rcw)
    return _final(acc2, h2, dis2, inv2, b2.reshape(1, D), Wfc,
                  bfc.reshape(1, DOUT))


# speedup vs baseline: 1.2564x; 1.0098x over previous
import functools

import jax
import jax.numpy as jnp
from jax import lax
from jax.experimental import pallas as pl
from jax.experimental.pallas import tpu as pltpu
from jax.experimental.pallas import tpu_sc as plsc

N = 10000
E = 320000
D = 128
DOUT = 64

NC = 2
NS = 16
NW = NC * NS
K = 125
NCH = (E // NW) // K
EPT_DEG = E // NS
NROW = N // 16

_MESH = plsc.VectorSubcoreMesh(core_axis_name="c", subcore_axis_name="s")


def _zero16():
    return jnp.zeros((16,), jnp.float32)


def _prep_body(col_h, ew_h, row_h, dix_h, w_h, dis_h, inv_h,
               colv, ewv, degf, buf625, invb, wbuf, dixv, deg_sh):
    c = lax.axis_index("c")
    s = lax.axis_index("s")
    wid = s * NC + c

    def zloop(i, _):
        degf[pl.ds(i * 16, 16)] = _zero16()
        buf625[i] = _zero16()
        return 0
    lax.fori_loop(0, NROW, zloop, 0)

    @pl.when(s == 0)
    def _():
        pltpu.sync_copy(buf625, deg_sh)

    pltpu.sync_copy(dix_h, dixv)

    base_e = s * EPT_DEG
    pltpu.sync_copy(col_h.at[pl.ds(base_e, EPT_DEG)], colv)
    pltpu.sync_copy(ew_h.at[pl.ds(base_e, EPT_DEG)], ewv)

    plsc.subcore_barrier()

    def dloop(i, _):
        off = i * 16
        cvec = colv[pl.ds(off, 16)]
        evec = ewv[pl.ds(off, 16)]
        plsc.addupdate_scatter(degf, [cvec], evec)
        return 0
    lax.fori_loop(0, EPT_DEG // 16, dloop, 0)

    def cpl(i, _):
        buf625[i] = degf[pl.ds(i * 16, 16)]
        return 0
    lax.fori_loop(0, NROW, cpl, 0)

    def closs(j, _):
        pltpu.sync_copy(buf625.at[pl.ds(j * K, K)],
                        deg_sh.at[dixv.at[j]], add=True)
        return 0
    lax.fori_loop(0, NROW // K, closs, 0)

    plsc.subcore_barrier()

    pltpu.sync_copy(deg_sh, buf625)

    def nloop(i, _):
        d = buf625[i] + 1.0
        iv = plsc.bitcast(d, jnp.int32)
        iv = jnp.int32(0x5F3759DF) - lax.shift_right_arithmetic(iv, 1)
        y = plsc.bitcast(iv, jnp.float32)
        y = y * (1.5 - 0.5 * d * y * y)
        y = y * (1.5 - 0.5 * d * y * y)
        y = y * (1.5 - 0.5 * d * y * y)
        buf625[i] = y
        degf[pl.ds(i * 16, 16)] = y
        return 0
    lax.fori_loop(0, NROW, nloop, 0)

    @pl.when(wid == 0)
    def _():
        def iloop(i, _):
            y = buf625[i]
            invb[i] = y * y
            return 0
        lax.fori_loop(0, NROW, iloop, 0)
        pltpu.sync_copy(buf625, dis_h)
        pltpu.sync_copy(invb, inv_h)

    base_w = wid * (E // NW)
    pltpu.sync_copy(row_h.at[pl.ds(base_w, E // NW)], colv.at[pl.ds(0, E // NW)])
    pltpu.sync_copy(ew_h.at[pl.ds(base_w, E // NW)], ewv.at[pl.ds(0, E // NW)])

    def wloop(i, _):
        off = i * 16
        rvec = colv[pl.ds(off, 16)]
        dvec = plsc.load_gather(degf, [rvec])
        wbuf[i] = dvec * ewv[pl.ds(off, 16)]
        return 0
    lax.fori_loop(0, (E // NW) // 16, wloop, 0)

    pltpu.sync_copy(wbuf, w_h.at[wid])


_prep = functools.partial(
    pl.kernel,
    out_type=[
        jax.ShapeDtypeStruct((NW, NROW, 16), jnp.float32),
        jax.ShapeDtypeStruct((NROW, 16), jnp.float32),
        jax.ShapeDtypeStruct((NROW, 16), jnp.float32),
    ],
    mesh=_MESH,
    compiler_params=pltpu.CompilerParams(needs_layout_passes=False, use_tc_tiling_on_sc=False),
    scratch_types=[
        pltpu.VMEM((EPT_DEG,), jnp.int32),
        pltpu.VMEM((EPT_DEG,), jnp.float32),
        pltpu.VMEM((N,), jnp.float32),
        pltpu.VMEM((NROW, 16), jnp.float32),
        pltpu.VMEM((NROW, 16), jnp.float32),
        pltpu.VMEM((NROW, 16), jnp.float32),
        pltpu.VMEM((NROW // K, K), jnp.int32),
        pltpu.MemorySpace.VMEM_SHARED((NROW, 16), jnp.float32),
    ],
)(_prep_body)


def _agg_body(h_h, rcw_h, acc_h, rcw0, rcw1, rcw2, rows0, rows1, rows2,
              sem0, sem1, sem2, ssem0, ssem1, ssem2, acc_sh):
    c = lax.axis_index("c")
    s = lax.axis_index("s")
    wid = s * NC + c

    def zloop(i, _):
        for t in range(8):
            rows2[i, pl.ds(t * 16, 16)] = _zero16()
        return 0
    lax.fori_loop(0, K, zloop, 0)

    base = s * (N // NS)

    def zs(j, _):
        pltpu.async_copy(rows2, acc_sh.at[pl.ds(base + j * K, K)], ssem2)
        return 0
    lax.fori_loop(0, (N // NS) // K, zs, 0)

    two = jnp.zeros((16,), jnp.int32) + 2

    def _scale(rows, rcw):
        def edge(i, _):
            for u in range(5):
                k = i * 5 + u
                kv = jnp.zeros((16,), jnp.int32) + k
                sv = plsc.bitcast(plsc.load_gather(rcw, [two, kv]), jnp.float32)
                for t in range(8):
                    sl = pl.ds(t * 16, 16)
                    rows[k, sl] = rows[k, sl] * sv
            return 0
        lax.fori_loop(0, K // 5, edge, 0)

    def _wait(rows, sem):
        pltpu.make_async_copy(h_h.at[pl.ds(0, K)], rows, sem).wait()

    bufs = ((rcw0, rows0, sem0, ssem0),
            (rcw1, rows1, sem1, ssem1),
            (rcw2, rows2, sem2, ssem2))

    def _proc(rcw, rows, gsem, ssm):
        _wait(rows, gsem)
        _scale(rows, rcw)
        pltpu.async_copy(rows, acc_sh.at[rcw.at[1]], ssm, add=True)

    def _refill(j2, rcw, rows, gsem, ssm, drain):
        if drain:
            _wait(rows, ssm)
        pltpu.sync_copy(rcw_h.at[wid, j2], rcw)
        pltpu.async_copy(h_h.at[rcw.at[0]], rows, gsem)

    pltpu.sync_copy(rcw_h.at[wid, 0], rcw0)
    pltpu.async_copy(h_h.at[rcw0.at[0]], rows0, sem0)
    pltpu.sync_copy(rcw_h.at[wid, 1], rcw1)
    pltpu.async_copy(h_h.at[rcw1.at[0]], rows1, sem1)

    def zdrain(j, _):
        _wait(rows2, ssem2)
        return 0
    lax.fori_loop(0, (N // NS) // K, zdrain, 0)

    plsc.subcore_barrier()

    _proc(*bufs[0])
    _refill(2, *bufs[2], drain=False)

    def body(i, _):
        for u in range(3):
            j = 3 * i + 1 + u
            b = (1 + u) % 3
            _proc(*bufs[b])
            rb = (b + 2) % 3

            @pl.when(j < NCH - 2)
            def _():
                _refill(j + 2, *bufs[rb], drain=True)
        return 0
    lax.fori_loop(0, (NCH - 2) // 3, body, 0)

    _proc(*bufs[(NCH - 1) % 3])
    _wait(rows0, ssem0)
    _wait(rows1, ssem1)
    _wait(rows2, ssem2)

    plsc.subcore_barrier()

    pltpu.sync_copy(acc_sh.at[pl.ds(base, N // NS)],
                    acc_h.at[c, pl.ds(base, N // NS)])


_agg = functools.partial(
    pl.kernel,
    out_type=jax.ShapeDtypeStruct((NC, N, D), jnp.float32),
    mesh=_MESH,
    compiler_params=pltpu.CompilerParams(needs_layout_passes=False, use_tc_tiling_on_sc=False),
    scratch_types=[
        pltpu.VMEM((3, K), jnp.int32),
        pltpu.VMEM((3, K), jnp.int32),
        pltpu.VMEM((3, K), jnp.int32),
        pltpu.VMEM((K, D), jnp.float32),
        pltpu.VMEM((K, D), jnp.float32),
        pltpu.VMEM((K, D), jnp.float32),
        pltpu.SemaphoreType.DMA,
        pltpu.SemaphoreType.DMA,
        pltpu.SemaphoreType.DMA,
        pltpu.SemaphoreType.DMA,
        pltpu.SemaphoreType.DMA,
        pltpu.SemaphoreType.DMA,
        pltpu.MemorySpace.VMEM_SHARED((N, D), jnp.float32),
    ],
)(_agg_body)


_RB = 1000


def _mm_body(x_ref, w_ref, o_ref):
    o_ref[...] = jnp.dot(x_ref[...], w_ref[...],
                         preferred_element_type=jnp.float32)


def _mm(x, W):
    return pl.pallas_call(
        _mm_body,
        grid=(N // _RB,),
        in_specs=[
            pl.BlockSpec((_RB, D), lambda i: (i, 0)),
            pl.BlockSpec((D, D), lambda i: (0, 0)),
        ],
        out_specs=pl.BlockSpec((_RB, D), lambda i: (i, 0)),
        out_shape=jax.ShapeDtypeStruct((N, D), jnp.float32),
    )(x, W)


def _emm_body(acc_ref, h_ref, dis_ref, inv_ref, b_ref, w_ref, o_ref):
    xn = jax.nn.relu(dis_ref[...] * (acc_ref[0] + acc_ref[1])
                     + inv_ref[...] * h_ref[...] + b_ref[...])
    o_ref[...] = jnp.dot(xn, w_ref[...], preferred_element_type=jnp.float32)


def _emm(acc, h, dis2, inv2, b, W):
    return pl.pallas_call(
        _emm_body,
        grid=(N // _RB,),
        in_specs=[
            pl.BlockSpec((NC, _RB, D), lambda i: (0, i, 0)),
            pl.BlockSpec((_RB, D), lambda i: (i, 0)),
            pl.BlockSpec((_RB, 1), lambda i: (i, 0)),
            pl.BlockSpec((_RB, 1), lambda i: (i, 0)),
            pl.BlockSpec((1, D), lambda i: (0, 0)),
            pl.BlockSpec((D, D), lambda i: (0, 0)),
        ],
        out_specs=pl.BlockSpec((_RB, D), lambda i: (i, 0)),
        out_shape=jax.ShapeDtypeStruct((N, D), jnp.float32),
    )(acc, h, dis2, inv2, b, W)


def _final_body(acc_ref, h_ref, dis_ref, inv_ref, b_ref, w_ref, bf_ref, o_ref):
    xn = jax.nn.relu(dis_ref[...] * (acc_ref[0] + acc_ref[1])
                     + inv_ref[...] * h_ref[...] + b_ref[...])
    logits = jnp.dot(xn, w_ref[...], preferred_element_type=jnp.float32)
    logits = logits + bf_ref[...]
    m = jnp.max(logits, axis=1, keepdims=True)
    sh = logits - m
    lse = jnp.log(jnp.sum(jnp.exp(sh), axis=1, keepdims=True))
    o_ref[...] = sh - lse


def _final(acc, h, dis2, inv2, b, Wfc, bfc):
    return pl.pallas_call(
        _final_body,
        grid=(N // _RB,),
        in_specs=[
            pl.BlockSpec((NC, _RB, D), lambda i: (0, i, 0)),
            pl.BlockSpec((_RB, D), lambda i: (i, 0)),
            pl.BlockSpec((_RB, 1), lambda i: (i, 0)),
            pl.BlockSpec((_RB, 1), lambda i: (i, 0)),
            pl.BlockSpec((1, D), lambda i: (0, 0)),
            pl.BlockSpec((D, DOUT), lambda i: (0, 0)),
            pl.BlockSpec((1, DOUT), lambda i: (0, 0)),
        ],
        out_specs=pl.BlockSpec((_RB, DOUT), lambda i: (i, 0)),
        out_shape=jax.ShapeDtypeStruct((N, DOUT), jnp.float32),
    )(acc, h, dis2, inv2, b, Wfc, bfc)


def kernel(x, edge_index, edge_attr, W1, b1, W2, b2, Wfc, bfc):
    row = edge_index[0].astype(jnp.int32)
    col = edge_index[1].astype(jnp.int32)
    ew = edge_attr.astype(jnp.float32)
    dix = jnp.arange(NROW, dtype=jnp.int32).reshape(NROW // K, K)

    h1 = _mm(x, W1)
    w3, dis625, inv625 = _prep(col, ew, row, dix)
    dis2 = dis625.reshape(N, 1)
    inv2 = inv625.reshape(N, 1)
    rcw = jnp.stack([row.reshape(NW, NCH, K),
                     col.reshape(NW, NCH, K),
                     lax.bitcast_convert_type(w3, jnp.int32).reshape(NW, NCH, K)],
                    axis=2)

    acc1 = _agg(h1, rcw)
    h2 = _emm(acc1, h1, dis2, inv2, b1.reshape(1, D), W2)
    acc2 = _agg(h2, rcw)
    return _final(acc2, h2, dis2, inv2, b2.reshape(1, D), Wfc,
                  bfc.reshape(1, DOUT))
